# baseline (speedup 1.0000x reference)
import functools

import jax
import jax.numpy as jnp
from jax import lax
from jax.experimental import pallas as pl
from jax.experimental.pallas import tpu as pltpu
from jax.experimental.pallas import tpu_sc as plsc

_NCLS = 10
_B = 1024
_NW = 32
_RING = 4


def _make_kernel():
    mesh = plsc.VectorSubcoreMesh(core_axis_name="c", subcore_axis_name="s")

    @functools.partial(
        pl.kernel,
        mesh=mesh,
        out_type=jax.ShapeDtypeStruct((_B, 100, 784), jnp.float32),
        scratch_types=[
            pltpu.VMEM((_B,), jnp.int32),
            pltpu.VMEM((1, 100, 784), jnp.float32),
            pltpu.SemaphoreType.DMA((_RING,)),
        ],
    )
    def emb(y_hbm, a_hbm, out_hbm, y_v, buf, sems):
        wid = lax.axis_index("s") * 2 + lax.axis_index("c")
        mycls = wid % _NCLS
        myrank = wid // _NCLS
        ntiles = jnp.where(mycls < _NW % _NCLS, _NW // _NCLS + 1, _NW // _NCLS)

        pltpu.sync_copy(y_hbm, y_v)
        pltpu.sync_copy(a_hbm.at[pl.ds(mycls, 1)], buf)

        def wrblock(i, slot):
            return pltpu.make_async_copy(buf, out_hbm.at[pl.ds(i, 1)],
                                         sems.at[slot])

        def block(g, carry):
            n, cnt = carry
            y16 = y_v[pl.ds(g * 16, 16)]
            for j in range(16):
                hit = y16[j] == mycls
                mine = hit & (lax.rem(cnt, ntiles) == myrank)
                slot = lax.rem(n, _RING)
                i = g * 16 + j

                @pl.when(mine & (n >= _RING))
                def _():
                    wrblock(i, slot).wait()

                @pl.when(mine)
                def _():
                    wrblock(i, slot).start()

                n = n + jnp.where(mine, 1, 0)
                cnt = cnt + jnp.where(hit, 1, 0)
            return n, cnt

        n, _ = lax.fori_loop(0, _B // 16, block, (jnp.int32(0), jnp.int32(0)))

        for s in range(_RING):
            @pl.when(n > s)
            def _():
                wrblock(0, s).wait()

    return emb


_emb = _make_kernel()


def kernel(y, A):
    a3 = A.reshape(_NCLS, 100, 784)
    return _emb(y.astype(jnp.int32), a3)
